# baseline (speedup 1.0000x reference)
import functools

import jax
import jax.numpy as jnp
from jax import lax
from jax.experimental import pallas as pl
from jax.experimental.pallas import tpu as pltpu
from jax.experimental.pallas import tpu_sc as plsc

N = 10000
E = 320000
D = 128
H = 2 * D
BN_EPS = 1e-5
QSCALE = 512.0

NC = 2
NS = 16
NW = NC * NS
C = 128
NBUF = 8
CHUNKS = 80
EPW = CHUNKS * C
E_PAD = NW * EPW
N_ACC = 10240
RPS = N_ACC // NS
TAIL = (E // NW) % C
NPAD = 2 * C - TAIL


def _agg_body(x_hbm, src_hbm, dst_hbm, z_hbm, out_hbm,
              src_v, dst_v, r0, r1, r2, r3, r4, r5, r6, r7, agg_sh,
              g0, g1, g2, g3, g4, g5, g6, g7,
              s0, s1, s2, s3, s4, s5, s6, s7):
    rows = (r0, r1, r2, r3, r4, r5, r6, r7)
    gsem = (g0, g1, g2, g3, g4, g5, g6, g7)
    ssem = (s0, s1, s2, s3, s4, s5, s6, s7)
    cid = lax.axis_index("c")
    sid = lax.axis_index("s")
    wid = cid * NS + sid

    with jax.named_scope("agg_prolog"):
        pltpu.sync_copy(z_hbm, agg_sh.at[pl.ds(sid * RPS, RPS)])
        pltpu.sync_copy(src_hbm.at[wid], src_v)
        pltpu.sync_copy(dst_hbm.at[wid], dst_v)
        plsc.subcore_barrier()

    for b in range(NBUF):
        pltpu.async_copy(x_hbm.at[src_v.at[b]], rows[b], gsem[b])

    @pl.loop(0, CHUNKS, step=NBUF)
    def _grp(g):
        sd = []
        for b in range(NBUF):
            i = g + b
            pltpu.make_async_copy(x_hbm.at[src_v.at[i]], rows[b],
                                  gsem[b]).wait()
            sd.append(pltpu.async_copy(rows[b], agg_sh.at[dst_v.at[i]],
                                       ssem[b], add=True))
        for b in range(NBUF):
            sd[b].wait()
            j = g + b + NBUF

            @pl.when(j < CHUNKS)
            def _(b=b, j=j):
                pltpu.async_copy(x_hbm.at[src_v.at[j]], rows[b], gsem[b])

    with jax.named_scope("agg_postbar"):
        plsc.subcore_barrier()
    base = sid * RPS

    @pl.when(sid < NS - 1)
    def _():
        pltpu.sync_copy(agg_sh.at[pl.ds(base, RPS)],
                        out_hbm.at[pl.ds(cid * N + base, RPS)])

    @pl.when(sid == NS - 1)
    def _():
        pltpu.sync_copy(agg_sh.at[pl.ds(base, N - (NS - 1) * RPS)],
                        out_hbm.at[pl.ds(cid * N + base, N - (NS - 1) * RPS)])


_agg = functools.partial(
    pl.kernel,
    out_type=jax.ShapeDtypeStruct((NC * N, D), jnp.int16),
    mesh=plsc.VectorSubcoreMesh(core_axis_name="c", subcore_axis_name="s",
                                num_cores=NC, num_subcores=NS),
    compiler_params=pltpu.CompilerParams(use_tc_tiling_on_sc=False),
    scratch_types=[
        pltpu.VMEM((CHUNKS, C), jnp.int32),
        pltpu.VMEM((CHUNKS, C), jnp.int32),
        pltpu.VMEM((C, D), jnp.int16),
        pltpu.VMEM((C, D), jnp.int16),
        pltpu.VMEM((C, D), jnp.int16),
        pltpu.VMEM((C, D), jnp.int16),
        pltpu.VMEM((C, D), jnp.int16),
        pltpu.VMEM((C, D), jnp.int16),
        pltpu.VMEM((C, D), jnp.int16),
        pltpu.VMEM((C, D), jnp.int16),
        pltpu.VMEM_SHARED((N_ACC, D), jnp.int16),
    ] + [pltpu.SemaphoreType.DMA] * (2 * NBUF),
)(_agg_body)


EN = E // NW
PADW = EPW - EN


def _prep_idx_body(s_ref, d_ref, so_ref, do_ref):
    pad_s = lax.broadcasted_iota(jnp.int32, (1, PADW), 1)
    so_ref[0] = jnp.concatenate([s_ref[0], pad_s], axis=1)
    do_ref[0] = jnp.concatenate([d_ref[0], pad_s + N], axis=1)


_prep_idx = pl.pallas_call(
    _prep_idx_body,
    grid=(NW,),
    in_specs=[
        pl.BlockSpec((1, 1, EN), lambda w: (w, 0, 0)),
        pl.BlockSpec((1, 1, EN), lambda w: (NW + w, 0, 0)),
    ],
    out_specs=[
        pl.BlockSpec((1, 1, EPW), lambda w: (w, 0, 0)),
        pl.BlockSpec((1, 1, EPW), lambda w: (w, 0, 0)),
    ],
    out_shape=[
        jax.ShapeDtypeStruct((NW, 1, EPW), jnp.int32),
        jax.ShapeDtypeStruct((NW, 1, EPW), jnp.int32),
    ],
)

RQ = 2000


def _quant_body(x_ref, q_ref):
    q_ref[...] = jnp.round(x_ref[...] * QSCALE).astype(jnp.int16)


_quant = pl.pallas_call(
    _quant_body,
    grid=(N // RQ,),
    in_specs=[pl.BlockSpec((RQ, D), lambda i: (i, 0))],
    out_specs=pl.BlockSpec((RQ, D), lambda i: (i, 0)),
    out_shape=jax.ShapeDtypeStruct((N, D), jnp.int16),
)


R = 2000
GB = N // R


def _mlp_body(scale_ref, x_ref, a_ref, w1_ref, b1_ref, gamma_ref,
              beta_ref, w2_ref, b2_ref, out_ref, h1_buf, sums_ref):
    p = pl.program_id(0)
    j = pl.program_id(1)

    @pl.when(p == 0)
    def _():
        agg = (a_ref[0].astype(jnp.float32)
               + a_ref[1].astype(jnp.float32)) * (1.0 / QSCALE)
        h = scale_ref[0, 0] * x_ref[...] + agg
        h1 = jnp.dot(h, w1_ref[...], preferred_element_type=jnp.float32)
        h1 = h1 + b1_ref[...]
        h1_buf[pl.ds(j * R, R), :] = h1

        @pl.when(j == 0)
        def _():
            sums_ref[...] = jnp.zeros_like(sums_ref)

        sums_ref[...] += jnp.stack([jnp.sum(h1, axis=0),
                                    jnp.sum(h1 * h1, axis=0)])

    @pl.when(p == 1)
    def _():
        mean = sums_ref[0:1, :] / N
        var = sums_ref[1:2, :] / N - mean * mean
        bscale = lax.rsqrt(var + BN_EPS) * gamma_ref[...]
        h1 = h1_buf[pl.ds(j * R, R), :]
        hn = (h1 - mean) * bscale + beta_ref[...]
        hn = jnp.maximum(hn, 0.0)
        out = jnp.dot(hn, w2_ref[...], preferred_element_type=jnp.float32)
        out_ref[...] = out + b2_ref[...]


def _row_map(p, j):
    return (jnp.where(p == 0, j, 0), 0)


_mlp = pl.pallas_call(
    _mlp_body,
    grid=(2, GB),
    in_specs=[
        pl.BlockSpec(memory_space=pltpu.SMEM),
        pl.BlockSpec((R, D), _row_map),
        pl.BlockSpec((NC, R, D), lambda p, j: (0, jnp.where(p == 0, j, 0), 0)),
        pl.BlockSpec((D, H), lambda p, j: (0, 0)),
        pl.BlockSpec((1, H), lambda p, j: (0, 0)),
        pl.BlockSpec((1, H), lambda p, j: (0, 0)),
        pl.BlockSpec((1, H), lambda p, j: (0, 0)),
        pl.BlockSpec((H, D), lambda p, j: (0, 0)),
        pl.BlockSpec((1, D), lambda p, j: (0, 0)),
    ],
    out_specs=pl.BlockSpec((R, D), lambda p, j: (jnp.where(p == 0, 0, j), 0)),
    out_shape=jax.ShapeDtypeStruct((N, D), jnp.float32),
    scratch_shapes=[pltpu.VMEM((N, H), jnp.float32),
                    pltpu.VMEM((2, H), jnp.float32)],
)


def kernel(x, edge_index, eps, W1, b1, gamma, beta, W2, b2):
    edge_r = edge_index.astype(jnp.int32).reshape(2 * NW, 1, EN)
    src_o, dst_o = _prep_idx(edge_r, edge_r)
    src3 = src_o.reshape(NW, CHUNKS, C)
    dst3 = dst_o.reshape(NW, CHUNKS, C)
    x_q = _quant(x)
    zeros_blk = jnp.zeros((RPS, D), jnp.int16)

    agg_flat = _agg(x_q, src3, dst3, zeros_blk)
    agg2 = agg_flat.reshape(NC, N, D)

    scale = jnp.reshape(1.0 + eps, (1, 1)).astype(jnp.float32)
    out = _mlp(scale, x, agg2, W1, b1.reshape(1, H), gamma.reshape(1, H),
               beta.reshape(1, H), W2, b2.reshape(1, D))
    return out


# speedup vs baseline: 1.0550x; 1.0550x over previous
import functools

import jax
import jax.numpy as jnp
from jax import lax
from jax.experimental import pallas as pl
from jax.experimental.pallas import tpu as pltpu
from jax.experimental.pallas import tpu_sc as plsc

N = 10000
E = 320000
D = 128
H = 2 * D
BN_EPS = 1e-5
QSCALE = 512.0

NC = 2
NS = 16
NW = NC * NS
C = 128
NBUF = 8
CHUNKS = 80
EPW = CHUNKS * C
E_PAD = NW * EPW
N_ACC = 10240
RPS = N_ACC // NS
TAIL = (E // NW) % C
NPAD = 2 * C - TAIL


def _agg_body(x_hbm, src_hbm, dst_hbm, z_hbm, out_hbm,
              src_v, dst_v, r0, r1, r2, r3, r4, r5, r6, r7, agg_sh,
              g0, g1, g2, g3, g4, g5, g6, g7,
              s0, s1, s2, s3, s4, s5, s6, s7):
    rows = (r0, r1, r2, r3, r4, r5, r6, r7)
    gsem = (g0, g1, g2, g3, g4, g5, g6, g7)
    ssem = (s0, s1, s2, s3, s4, s5, s6, s7)
    cid = lax.axis_index("c")
    sid = lax.axis_index("s")
    wid = cid * NS + sid

    ds = [pltpu.async_copy(z_hbm, agg_sh.at[pl.ds(sid * RPS, RPS)], gsem[0]),
          pltpu.async_copy(src_hbm.at[wid], src_v, gsem[1]),
          pltpu.async_copy(dst_hbm.at[wid], dst_v, gsem[2])]
    for d in ds:
        d.wait()
    plsc.subcore_barrier()

    for b in range(NBUF):
        pltpu.async_copy(x_hbm.at[src_v.at[b]], rows[b], gsem[b])

    @pl.loop(0, CHUNKS, step=NBUF)
    def _grp(g):
        sd = []
        for b in range(NBUF):
            i = g + b
            pltpu.make_async_copy(x_hbm.at[src_v.at[i]], rows[b],
                                  gsem[b]).wait()
            sd.append(pltpu.async_copy(rows[b], agg_sh.at[dst_v.at[i]],
                                       ssem[b], add=True))
        for b in range(NBUF):
            sd[b].wait()
            j = g + b + NBUF

            @pl.when(j < CHUNKS)
            def _(b=b, j=j):
                pltpu.async_copy(x_hbm.at[src_v.at[j]], rows[b], gsem[b])

    plsc.subcore_barrier()
    base = sid * RPS

    @pl.when(sid < NS - 1)
    def _():
        pltpu.sync_copy(agg_sh.at[pl.ds(base, RPS)],
                        out_hbm.at[pl.ds(cid * N + base, RPS)])

    @pl.when(sid == NS - 1)
    def _():
        pltpu.sync_copy(agg_sh.at[pl.ds(base, N - (NS - 1) * RPS)],
                        out_hbm.at[pl.ds(cid * N + base, N - (NS - 1) * RPS)])


_agg = functools.partial(
    pl.kernel,
    out_type=jax.ShapeDtypeStruct((NC * N, D), jnp.int16),
    mesh=plsc.VectorSubcoreMesh(core_axis_name="c", subcore_axis_name="s",
                                num_cores=NC, num_subcores=NS),
    compiler_params=pltpu.CompilerParams(use_tc_tiling_on_sc=False),
    scratch_types=[
        pltpu.VMEM((CHUNKS, C), jnp.int32),
        pltpu.VMEM((CHUNKS, C), jnp.int32),
        pltpu.VMEM((C, D), jnp.int16),
        pltpu.VMEM((C, D), jnp.int16),
        pltpu.VMEM((C, D), jnp.int16),
        pltpu.VMEM((C, D), jnp.int16),
        pltpu.VMEM((C, D), jnp.int16),
        pltpu.VMEM((C, D), jnp.int16),
        pltpu.VMEM((C, D), jnp.int16),
        pltpu.VMEM((C, D), jnp.int16),
        pltpu.VMEM_SHARED((N_ACC, D), jnp.int16),
    ] + [pltpu.SemaphoreType.DMA] * (2 * NBUF),
)(_agg_body)


R = 2000
GB = N // R


def _mlp_body(scale_ref, x_ref, a_ref, w1_ref, b1_ref, gamma_ref,
              beta_ref, w2_ref, b2_ref, out_ref, h1_buf, sums_ref):
    p = pl.program_id(0)
    j = pl.program_id(1)

    @pl.when(p == 0)
    def _():
        agg = (a_ref[0].astype(jnp.float32)
               + a_ref[1].astype(jnp.float32)) * (1.0 / QSCALE)
        h = scale_ref[0, 0] * x_ref[...] + agg
        h1 = jnp.dot(h, w1_ref[...], preferred_element_type=jnp.float32)
        h1 = h1 + b1_ref[...]
        h1_buf[pl.ds(j * R, R), :] = h1

        @pl.when(j == 0)
        def _():
            sums_ref[...] = jnp.zeros_like(sums_ref)

        sums_ref[...] += jnp.stack([jnp.sum(h1, axis=0),
                                    jnp.sum(h1 * h1, axis=0)])

    @pl.when(p == 1)
    def _():
        mean = sums_ref[0:1, :] / N
        var = sums_ref[1:2, :] / N - mean * mean
        bscale = lax.rsqrt(var + BN_EPS) * gamma_ref[...]
        h1 = h1_buf[pl.ds(j * R, R), :]
        hn = (h1 - mean) * bscale + beta_ref[...]
        hn = jnp.maximum(hn, 0.0)
        out = jnp.dot(hn, w2_ref[...], preferred_element_type=jnp.float32)
        out_ref[...] = out + b2_ref[...]


def _row_map(p, j):
    return (jnp.where(p == 0, j, 0), 0)


_mlp = pl.pallas_call(
    _mlp_body,
    grid=(2, GB),
    in_specs=[
        pl.BlockSpec(memory_space=pltpu.SMEM),
        pl.BlockSpec((R, D), _row_map),
        pl.BlockSpec((NC, R, D), lambda p, j: (0, jnp.where(p == 0, j, 0), 0)),
        pl.BlockSpec((D, H), lambda p, j: (0, 0)),
        pl.BlockSpec((1, H), lambda p, j: (0, 0)),
        pl.BlockSpec((1, H), lambda p, j: (0, 0)),
        pl.BlockSpec((1, H), lambda p, j: (0, 0)),
        pl.BlockSpec((H, D), lambda p, j: (0, 0)),
        pl.BlockSpec((1, D), lambda p, j: (0, 0)),
    ],
    out_specs=pl.BlockSpec((R, D), lambda p, j: (jnp.where(p == 0, 0, j), 0)),
    out_shape=jax.ShapeDtypeStruct((N, D), jnp.float32),
    scratch_shapes=[pltpu.VMEM((N, H), jnp.float32),
                    pltpu.VMEM((2, H), jnp.float32)],
)


def kernel(x, edge_index, eps, W1, b1, gamma, beta, W2, b2):
    src = edge_index[0].astype(jnp.int32)
    dst = edge_index[1].astype(jnp.int32)
    padw = EPW - E // NW
    pad_src = jnp.broadcast_to(jnp.arange(padw, dtype=jnp.int32)[None],
                               (NW, padw))
    pad_dst = jnp.broadcast_to(N + jnp.arange(padw, dtype=jnp.int32)[None],
                               (NW, padw))
    src3 = jnp.concatenate([src.reshape(NW, E // NW), pad_src],
                           axis=1).reshape(NW, CHUNKS, C)
    dst3 = jnp.concatenate([dst.reshape(NW, E // NW), pad_dst],
                           axis=1).reshape(NW, CHUNKS, C)
    x_q = jnp.round(x * QSCALE).astype(jnp.int16)
    zeros_blk = jnp.zeros((RPS, D), jnp.int16)

    agg_flat = _agg(x_q, src3, dst3, zeros_blk)
    agg2 = agg_flat.reshape(NC, N, D)

    scale = jnp.reshape(1.0 + eps, (1, 1)).astype(jnp.float32)
    out = _mlp(scale, x, agg2, W1, b1.reshape(1, H), gamma.reshape(1, H),
               beta.reshape(1, H), W2, b2.reshape(1, D))
    return out


# speedup vs baseline: 1.0560x; 1.0010x over previous
import functools

import jax
import jax.numpy as jnp
from jax import lax
from jax.experimental import pallas as pl
from jax.experimental.pallas import tpu as pltpu
from jax.experimental.pallas import tpu_sc as plsc

N = 10000
E = 320000
D = 128
H = 2 * D
BN_EPS = 1e-5
QSCALE = 512.0

NC = 2
NS = 16
NW = NC * NS
C = 128
NBUF = 8
CHUNKS = 80
EPW = CHUNKS * C
N_ACC = 10240
RPS = N_ACC // NS


def _agg_body(x_hbm, src_hbm, dst_hbm, z_hbm, out_hbm,
              src_v, dst_v, r0, r1, r2, r3, r4, r5, r6, r7, agg_sh,
              g0, g1, g2, g3, g4, g5, g6, g7,
              s0, s1, s2, s3, s4, s5, s6, s7):
    rows = (r0, r1, r2, r3, r4, r5, r6, r7)
    gsem = (g0, g1, g2, g3, g4, g5, g6, g7)
    ssem = (s0, s1, s2, s3, s4, s5, s6, s7)
    cid = lax.axis_index("c")
    sid = lax.axis_index("s")
    wid = cid * NS + sid

    ds = [pltpu.async_copy(z_hbm, agg_sh.at[pl.ds(sid * RPS, RPS)], gsem[0]),
          pltpu.async_copy(src_hbm.at[wid], src_v, gsem[1]),
          pltpu.async_copy(dst_hbm.at[wid], dst_v, gsem[2])]
    for d in ds:
        d.wait()
    plsc.subcore_barrier()

    for b in range(NBUF):
        pltpu.async_copy(x_hbm.at[src_v.at[b]], rows[b], gsem[b])

    @pl.loop(0, CHUNKS, step=NBUF)
    def _grp(g):
        sd = []
        for b in range(NBUF):
            i = g + b
            pltpu.make_async_copy(x_hbm.at[src_v.at[i]], rows[b],
                                  gsem[b]).wait()
            sd.append(pltpu.async_copy(rows[b], agg_sh.at[dst_v.at[i]],
                                       ssem[b], add=True))
        for b in range(NBUF):
            sd[b].wait()
            j = g + b + NBUF

            @pl.when(j < CHUNKS)
            def _(b=b, j=j):
                pltpu.async_copy(x_hbm.at[src_v.at[j]], rows[b], gsem[b])

    plsc.subcore_barrier()
    base = sid * RPS

    @pl.when(sid < NS - 1)
    def _():
        pltpu.sync_copy(agg_sh.at[pl.ds(base, RPS)],
                        out_hbm.at[pl.ds(cid * N + base, RPS)])

    @pl.when(sid == NS - 1)
    def _():
        pltpu.sync_copy(agg_sh.at[pl.ds(base, N - (NS - 1) * RPS)],
                        out_hbm.at[pl.ds(cid * N + base, N - (NS - 1) * RPS)])


_agg = functools.partial(
    pl.kernel,
    out_type=jax.ShapeDtypeStruct((NC * N, D), jnp.int16),
    mesh=plsc.VectorSubcoreMesh(core_axis_name="c", subcore_axis_name="s",
                                num_cores=NC, num_subcores=NS),
    compiler_params=pltpu.CompilerParams(use_tc_tiling_on_sc=False),
    scratch_types=[
        pltpu.VMEM((CHUNKS, C), jnp.int32),
        pltpu.VMEM((CHUNKS, C), jnp.int32),
        pltpu.VMEM((C, D), jnp.int16),
        pltpu.VMEM((C, D), jnp.int16),
        pltpu.VMEM((C, D), jnp.int16),
        pltpu.VMEM((C, D), jnp.int16),
        pltpu.VMEM((C, D), jnp.int16),
        pltpu.VMEM((C, D), jnp.int16),
        pltpu.VMEM((C, D), jnp.int16),
        pltpu.VMEM((C, D), jnp.int16),
        pltpu.VMEM_SHARED((N_ACC, D), jnp.int16),
    ] + [pltpu.SemaphoreType.DMA] * (2 * NBUF),
)(_agg_body)


R = 2000
GB = N // R


def _mlp_body(scale_ref, x_ref, a_ref, w1_ref, b1_ref, gamma_ref,
              beta_ref, w2_ref, b2_ref, out_ref, h1_buf, sums_ref):
    p = pl.program_id(0)
    j = pl.program_id(1)

    @pl.when(p == 0)
    def _():
        agg = (a_ref[0].astype(jnp.float32)
               + a_ref[1].astype(jnp.float32)) * (1.0 / QSCALE)
        h = scale_ref[0, 0] * x_ref[...] + agg
        h1 = jnp.dot(h, w1_ref[...], preferred_element_type=jnp.float32)
        h1 = h1 + b1_ref[...]
        h1_buf[pl.ds(j * R, R), :] = h1

        @pl.when(j == 0)
        def _():
            sums_ref[...] = jnp.zeros_like(sums_ref)

        sums_ref[...] += jnp.stack([jnp.sum(h1, axis=0),
                                    jnp.sum(h1 * h1, axis=0)])

    @pl.when(p == 1)
    def _():
        mean = sums_ref[0:1, :] / N
        var = sums_ref[1:2, :] / N - mean * mean
        bscale = lax.rsqrt(var + BN_EPS) * gamma_ref[...]
        h1 = h1_buf[pl.ds(j * R, R), :]
        hn = (h1 - mean) * bscale + beta_ref[...]
        hn = jnp.maximum(hn, 0.0)
        out = jnp.dot(hn, w2_ref[...], preferred_element_type=jnp.float32)
        out_ref[...] = out + b2_ref[...]


def _row_map(p, j):
    return (jnp.where(p == 0, j, 0), 0)


_mlp = pl.pallas_call(
    _mlp_body,
    grid=(2, GB),
    in_specs=[
        pl.BlockSpec(memory_space=pltpu.SMEM),
        pl.BlockSpec((R, D), _row_map),
        pl.BlockSpec((NC, R, D), lambda p, j: (0, jnp.where(p == 0, j, 0), 0)),
        pl.BlockSpec((D, H), lambda p, j: (0, 0)),
        pl.BlockSpec((1, H), lambda p, j: (0, 0)),
        pl.BlockSpec((1, H), lambda p, j: (0, 0)),
        pl.BlockSpec((1, H), lambda p, j: (0, 0)),
        pl.BlockSpec((H, D), lambda p, j: (0, 0)),
        pl.BlockSpec((1, D), lambda p, j: (0, 0)),
    ],
    out_specs=pl.BlockSpec((R, D), lambda p, j: (jnp.where(p == 0, 0, j), 0)),
    out_shape=jax.ShapeDtypeStruct((N, D), jnp.float32),
    scratch_shapes=[pltpu.VMEM((N, H), jnp.float32),
                    pltpu.VMEM((2, H), jnp.float32)],
)


def kernel(x, edge_index, eps, W1, b1, gamma, beta, W2, b2):
    src = edge_index[0].astype(jnp.int32)
    dst = edge_index[1].astype(jnp.int32)
    padw = EPW - E // NW
    pad_src = jnp.broadcast_to(jnp.arange(padw, dtype=jnp.int32)[None],
                               (NW, padw))
    pad_dst = jnp.broadcast_to(N + jnp.arange(padw, dtype=jnp.int32)[None],
                               (NW, padw))
    src3 = jnp.concatenate([src.reshape(NW, E // NW), pad_src],
                           axis=1).reshape(NW, CHUNKS, C)
    dst3 = jnp.concatenate([dst.reshape(NW, E // NW), pad_dst],
                           axis=1).reshape(NW, CHUNKS, C)
    x_q = jnp.round(x * QSCALE).astype(jnp.int16)
    zeros_blk = jnp.zeros((RPS, D), jnp.int16)

    agg_flat = _agg(x_q, src3, dst3, zeros_blk)
    agg2 = agg_flat.reshape(NC, N, D)

    scale = jnp.reshape(1.0 + eps, (1, 1)).astype(jnp.float32)
    out = _mlp(scale, x, agg2, W1, b1.reshape(1, H), gamma.reshape(1, H),
               beta.reshape(1, H), W2, b2.reshape(1, D))
    return out
